# baseline (speedup 1.0000x reference)
import functools
import math

import jax
import jax.numpy as jnp
import numpy as np
from jax import lax
from jax.experimental import pallas as pl
from jax.experimental.pallas import tpu as pltpu
from jax.experimental.pallas import tpu_sc as plsc

_BIG = 1e30


def _bin_transform():
    p = np.array([[0.0, -0.5, 0.5], [1.0, 0.0, -1.0], [0.0, 0.5, 0.5]])
    t27 = np.kron(np.kron(p, p), p)
    t81 = np.zeros((81, 81), np.float32)
    for s in range(3):
        t81[27 * s:27 * (s + 1), 27 * s:27 * (s + 1)] = t27
    return t81


_T81 = _bin_transform()


def _pack_bf16_pair(a, b):
    ai = jax.lax.bitcast_convert_type(a, jnp.int32)
    bi = jax.lax.bitcast_convert_type(b, jnp.int32)
    ar = ai + 0x7FFF + ((ai >> 16) & 1)
    br = bi + 0x7FFF + ((bi >> 16) & 1)
    packed = (ar & np.int32(-65536)) | ((br >> 16) & 0xFFFF)
    return jax.lax.bitcast_convert_type(packed, jnp.float32)


def _unpack_bf16_pair(p):
    pi = jax.lax.bitcast_convert_type(p, jnp.int32)
    a = jax.lax.bitcast_convert_type(pi & np.int32(-65536), jnp.float32)
    b = jax.lax.bitcast_convert_type(pi << 16, jnp.float32)
    return a, b


def _corr_body(f1_ref, f2_ref, out_ref):
    d = f1_ref.shape[1]
    out_ref[0] = jax.lax.dot_general(
        f1_ref[0], f2_ref[0], (((0,), (0,)), ((), ())),
        preferred_element_type=jnp.float32,
        precision=jax.lax.Precision.HIGHEST) * (1.0 / math.sqrt(d))


def _round_rne(x):
    return (x + 12582912.0) - 12582912.0


def _sc_voxel_call(corr_nm, coords_flat, c2t_flat, rows, m):
    nw = 32
    rpw = rows // nw
    ch = 64
    nch = rpw // ch
    mesh = plsc.VectorSubcoreMesh(core_axis_name="c", subcore_axis_name="s")

    @functools.partial(
        pl.kernel, mesh=mesh,
        out_type=jax.ShapeDtypeStruct((rows, 96), jnp.float32),
        compiler_params=pltpu.CompilerParams(needs_layout_passes=False),
        scratch_types=[
            pltpu.VMEM((ch, m), jnp.float32),
            pltpu.VMEM((rpw * 3 + 16,), jnp.float32),
            pltpu.VMEM((3 * m,), jnp.float32),
            pltpu.VMEM((ch, 96), jnp.float32),
            pltpu.VMEM((96,), jnp.float32),
            pltpu.VMEM((96,), jnp.float32),
        ],
    )
    def k(corr_hbm, coords_hbm, c2_hbm, out_hbm, corr_v, cd_v, c2_v,
          ob_v, acc_v, cnt_v):
        wid = lax.axis_index("s") * 2 + lax.axis_index("c")
        bi = (wid * rpw) // (rows // 2)
        pltpu.sync_copy(coords_hbm.at[pl.ds(wid * rpw * 3, rpw * 3)],
                        cd_v.at[pl.ds(0, rpw * 3)])
        pltpu.sync_copy(c2_hbm.at[pl.ds(bi * 3 * m, 3 * m)], c2_v)
        ones16 = jnp.full((16,), 1.0, jnp.float32)
        zeros16 = jnp.zeros((16,), jnp.float32)
        for c6 in range(6):
            acc_v[pl.ds(16 * c6, 16)] = zeros16
            cnt_v[pl.ds(16 * c6, 16)] = zeros16

        def do_chunk(g, _):
            rowbase = wid * rpw + g * ch
            pltpu.sync_copy(corr_hbm.at[pl.ds(rowbase, ch), :], corr_v)

            def do_row(r, _):
                off = (g * ch + r) * 3
                cvec = cd_v[pl.ds(off, 16)]
                cx = jnp.full((16,), cvec[0], jnp.float32)
                cy = jnp.full((16,), cvec[1], jnp.float32)
                cz = jnp.full((16,), cvec[2], jnp.float32)

                def do_mchunk(j, _):
                    sl = pl.ds(j * 16, 16)
                    cj = corr_v[r, sl]
                    relx = c2_v[pl.ds(j * 16, 16)] - cx
                    rely = c2_v[pl.ds(m + j * 16, 16)] - cy
                    relz = c2_v[pl.ds(2 * m + j * 16, 16)] - cz
                    for si, s in enumerate((4.0, 2.0, 1.0)):
                        dvx = _round_rne(relx * s)
                        dvy = _round_rne(rely * s)
                        dvz = _round_rne(relz * s)
                        valid = ((jnp.abs(dvx) <= 1.0)
                                 & (jnp.abs(dvy) <= 1.0)
                                 & (jnp.abs(dvz) <= 1.0))
                        idxf = dvx * 9.0 + dvy * 3.0 + dvz + (13.0 + 27.0 * si)
                        idx = jnp.where(valid, idxf, 0.0).astype(jnp.int32)
                        plsc.addupdate_scatter(acc_v, [idx], cj, mask=valid)
                        plsc.addupdate_scatter(cnt_v, [idx], ones16,
                                               mask=valid)
                    return 0

                lax.fori_loop(0, m // 16, do_mchunk, 0)
                for c6 in range(6):
                    sl = pl.ds(16 * c6, 16)
                    a = acc_v[sl]
                    c = cnt_v[sl]
                    ob_v[r, sl] = a / jnp.maximum(c, 1.0)
                    acc_v[sl] = zeros16
                    cnt_v[sl] = zeros16
                return 0

            lax.fori_loop(0, ch, do_row, 0)
            pltpu.sync_copy(ob_v, out_hbm.at[pl.ds(rowbase, ch), :])
            return 0

        lax.fori_loop(0, nch, do_chunk, 0)

    return k(corr_nm, coords_flat, c2t_flat)


def _stage1_body(n_p2, knn, f1_ref, f2_ref, ct_ref, c2_ref, kw_ref, kb_ref,
                 zmax_ref, zmin_ref, zsum_ref, zsq_ref):
    f1 = f1_ref[0]
    f2 = f2_ref[0]
    d = f1.shape[0]
    n = f1.shape[1]
    m = n_p2
    corr = jax.lax.dot_general(
        f2, f1, (((0,), (0,)), ((), ())),
        preferred_element_type=jnp.float32,
        precision=jax.lax.Precision.HIGHEST) * (1.0 / math.sqrt(d))

    cx = ct_ref[0, 0:1, :]
    cy = ct_ref[0, 1:2, :]
    cz = ct_ref[0, 2:3, :]
    c2x = c2_ref[0, :, 0:1]
    c2y = c2_ref[0, :, 1:2]
    c2z = c2_ref[0, :, 2:3]
    relx = c2x - cx
    rely = c2y - cy
    relz = c2z - cz

    dist = relx * relx + rely * rely + relz * relz
    iota = jax.lax.broadcasted_iota(jnp.int32, (m, n), 0)
    key0 = ((jax.lax.bitcast_convert_type(dist, jnp.int32)
             & np.int32(-512)) | iota)
    p1 = _pack_bf16_pair(corr, relx)
    p2 = _pack_bf16_pair(rely, relz)
    h = m // 2
    ka = key0[0:h, :]
    kb_ = key0[h:m, :]
    sw = ka <= kb_
    kcur0 = jnp.where(sw, ka, kb_)
    kalt0 = jnp.where(sw, kb_, ka)
    p1cur0 = jnp.where(sw, p1[0:h, :], p1[h:m, :])
    p1alt = jnp.where(sw, p1[h:m, :], p1[0:h, :])
    p2cur0 = jnp.where(sw, p2[0:h, :], p2[h:m, :])
    p2alt = jnp.where(sw, p2[h:m, :], p2[0:h, :])
    kw0 = kw_ref[:, 0:1]
    kw1 = kw_ref[:, 1:2]
    kw2 = kw_ref[:, 2:3]
    kw3 = kw_ref[:, 3:4]
    kb = kb_ref[:, 0:1]

    def body(_, carry):
        kcur, kalt, p1cur, p2cur, zmax, zmin, zs, zq = carry
        mn = jnp.min(kcur, axis=0, keepdims=True)
        fm = kcur == mn
        s1 = jnp.sum(jnp.where(fm, p1cur, 0.0), axis=0, keepdims=True)
        s2 = jnp.sum(jnp.where(fm, p2cur, 0.0), axis=0, keepdims=True)
        csel, rxs = _unpack_bf16_pair(s1)
        rys, rzs = _unpack_bf16_pair(s2)
        z = kw0 * csel + kw1 * rxs + kw2 * rys + kw3 * rzs + kb
        kcur = jnp.where(fm, kalt, kcur)
        kalt = jnp.where(fm, jnp.int32(2 ** 31 - 1), kalt)
        p1cur = jnp.where(fm, p1alt, p1cur)
        p2cur = jnp.where(fm, p2alt, p2cur)
        return (kcur, kalt, p1cur, p2cur,
                jnp.maximum(zmax, z), jnp.minimum(zmin, z),
                zs + z, zq + z * z)

    z0 = jnp.zeros((64, n), jnp.float32)
    init = (kcur0, kalt0, p1cur0, p2cur0,
            jnp.full((64, n), -_BIG, jnp.float32),
            jnp.full((64, n), _BIG, jnp.float32), z0, z0)
    _, _, _, _, zmax, zmin, zs, zq = jax.lax.fori_loop(0, knn, body, init)
    zmax_ref[0] = zmax
    zmin_ref[0] = zmin

    ib = pl.program_id(1)
    ps = jnp.sum(zs, axis=1, keepdims=True)
    pq = jnp.sum(zq, axis=1, keepdims=True)

    @pl.when(ib == 0)
    def _():
        zsum_ref[0] = ps
        zsq_ref[0] = pq

    @pl.when(ib != 0)
    def _():
        zsum_ref[0] += ps
        zsq_ref[0] += pq


def _stage2_body(n_p, knn, vox_ref, zmax_ref, zmin_ref, zsum_ref, zsq_ref,
                 w1_ref, b1_ref, g1_ref, be1_ref, a1_ref, w2_ref, b2_ref,
                 kg_ref, kbe_ref, ka_ref, ow_ref, ob_ref, out_ref):
    eps = jnp.float32(1e-5)
    vox = vox_ref[0][:, 0:81]
    x = jax.lax.dot_general(
        w1_ref[...], vox, (((1,), (1,)), ((), ())),
        preferred_element_type=jnp.float32,
        precision=jax.lax.Precision.HIGHEST) + b1_ref[...]
    chs = jnp.sum(x, axis=1, keepdims=True)
    chq = jnp.sum(x * x, axis=1, keepdims=True)
    gi = jax.lax.broadcasted_iota(jnp.int32, (8, 128), 0)
    ci = jax.lax.broadcasted_iota(jnp.int32, (8, 128), 1)
    gmat = ((ci // 16) == gi).astype(jnp.float32)
    emat = gmat.T
    cnt1 = jnp.float32(16 * n_p)
    gmean = jax.lax.dot_general(gmat, chs, (((1,), (0,)), ((), ())),
                                preferred_element_type=jnp.float32) / cnt1
    gsq = jax.lax.dot_general(gmat, chq, (((1,), (0,)), ((), ())),
                              preferred_element_type=jnp.float32) / cnt1
    gvar = gsq - gmean * gmean
    mean_c = jax.lax.dot_general(emat, gmean, (((1,), (0,)), ((), ())),
                                 preferred_element_type=jnp.float32)
    var_c = jax.lax.dot_general(emat, gvar, (((1,), (0,)), ((), ())),
                                preferred_element_type=jnp.float32)
    xn = (x - mean_c) * (g1_ref[...] * jax.lax.rsqrt(var_c + eps)) + be1_ref[...]
    xa = jnp.where(xn >= 0.0, xn, a1_ref[0, 0] * xn)
    vol_out = jax.lax.dot_general(
        w2_ref[...], xa, (((1,), (0,)), ((), ())),
        preferred_element_type=jnp.float32,
        precision=jax.lax.Precision.HIGHEST) + b2_ref[...]

    zsum = zsum_ref[0]
    zsq = zsq_ref[0]
    gi2 = jax.lax.broadcasted_iota(jnp.int32, (8, 64), 0)
    ci2 = jax.lax.broadcasted_iota(jnp.int32, (8, 64), 1)
    gmat2 = ((ci2 // 8) == gi2).astype(jnp.float32)
    emat2 = gmat2.T
    cnt2 = jnp.float32(8 * n_p * knn)
    gmean2 = jax.lax.dot_general(gmat2, zsum, (((1,), (0,)), ((), ())),
                                 preferred_element_type=jnp.float32) / cnt2
    gsq2 = jax.lax.dot_general(gmat2, zsq, (((1,), (0,)), ((), ())),
                               preferred_element_type=jnp.float32) / cnt2
    gvar2 = gsq2 - gmean2 * gmean2
    mean_k = jax.lax.dot_general(emat2, gmean2, (((1,), (0,)), ((), ())),
                                 preferred_element_type=jnp.float32)
    var_k = jax.lax.dot_general(emat2, gvar2, (((1,), (0,)), ((), ())),
                                preferred_element_type=jnp.float32)
    slope = kg_ref[...] * jax.lax.rsqrt(var_k + eps)
    shift = kbe_ref[...] - mean_k * slope
    zpick = jnp.where(slope >= 0.0, zmax_ref[0], zmin_ref[0])
    kf = slope * zpick + shift
    kf = jnp.where(kf >= 0.0, kf, ka_ref[0, 0] * kf)
    knn_out = jax.lax.dot_general(
        ow_ref[...], kf, (((1,), (0,)), ((), ())),
        preferred_element_type=jnp.float32,
        precision=jax.lax.Precision.HIGHEST) + ob_ref[...]
    out_ref[0] = vol_out + knn_out


def _pvx_forward(coords, coords2, fmap1, fmap2, vol_w1, vol_b1, vol_g1,
                 vol_be1, vol_a, vol_w2, vol_b2, knn_w, knn_b, knn_g, knn_be,
                 knn_a, out_w, out_b, *, block_n=512, interpret=False):
    b, n_p, _ = coords.shape
    n_p2 = coords2.shape[1]
    d = fmap1.shape[1]
    knn = 32
    block_n = min(block_n, n_p)
    nb = n_p // block_n
    coords_t = jnp.transpose(coords, (0, 2, 1))
    kb2 = knn_b.reshape(64, 1)

    cb = 512
    corr_nm = pl.pallas_call(
        _corr_body,
        grid=(b, n_p // cb),
        in_specs=[
            pl.BlockSpec((1, d, cb), lambda i, j: (i, 0, j)),
            pl.BlockSpec((1, d, n_p2), lambda i, j: (i, 0, 0)),
        ],
        out_specs=pl.BlockSpec((1, cb, n_p2), lambda i, j: (i, j, 0)),
        out_shape=jax.ShapeDtypeStruct((b, n_p, n_p2), jnp.float32),
        interpret=interpret,
    )(fmap1, fmap2)

    rows = b * n_p
    vox = _sc_voxel_call(corr_nm.reshape(rows, n_p2),
                         coords.reshape(rows * 3),
                         jnp.transpose(coords2, (0, 2, 1)).reshape(b * 3 * n_p2),
                         rows, n_p2)
    vox = vox.reshape(b, n_p, 96)

    grid1 = (b, nb)
    zmax, zmin, zsum, zsq = pl.pallas_call(
        functools.partial(_stage1_body, n_p2, knn),
        grid=grid1,
        in_specs=[
            pl.BlockSpec((1, d, block_n), lambda i, j: (i, 0, j)),
            pl.BlockSpec((1, d, n_p2), lambda i, j: (i, 0, 0)),
            pl.BlockSpec((1, 3, block_n), lambda i, j: (i, 0, j)),
            pl.BlockSpec((1, n_p2, 3), lambda i, j: (i, 0, 0)),
            pl.BlockSpec((64, 4), lambda i, j: (0, 0)),
            pl.BlockSpec((64, 1), lambda i, j: (0, 0)),
        ],
        out_specs=[
            pl.BlockSpec((1, 64, block_n), lambda i, j: (i, 0, j)),
            pl.BlockSpec((1, 64, block_n), lambda i, j: (i, 0, j)),
            pl.BlockSpec((1, 64, 1), lambda i, j: (i, 0, 0)),
            pl.BlockSpec((1, 64, 1), lambda i, j: (i, 0, 0)),
        ],
        out_shape=[
            jax.ShapeDtypeStruct((b, 64, n_p), jnp.float32),
            jax.ShapeDtypeStruct((b, 64, n_p), jnp.float32),
            jax.ShapeDtypeStruct((b, 64, 1), jnp.float32),
            jax.ShapeDtypeStruct((b, 64, 1), jnp.float32),
        ],
        interpret=interpret,
    )(fmap1, fmap2, coords_t, coords2, knn_w, kb2)

    out = pl.pallas_call(
        functools.partial(_stage2_body, n_p, knn),
        grid=(b,),
        in_specs=[
            pl.BlockSpec((1, n_p, 96), lambda i: (i, 0, 0)),
            pl.BlockSpec((1, 64, n_p), lambda i: (i, 0, 0)),
            pl.BlockSpec((1, 64, n_p), lambda i: (i, 0, 0)),
            pl.BlockSpec((1, 64, 1), lambda i: (i, 0, 0)),
            pl.BlockSpec((1, 64, 1), lambda i: (i, 0, 0)),
            pl.BlockSpec((128, 81), lambda i: (0, 0)),
            pl.BlockSpec((128, 1), lambda i: (0, 0)),
            pl.BlockSpec((128, 1), lambda i: (0, 0)),
            pl.BlockSpec((128, 1), lambda i: (0, 0)),
            pl.BlockSpec((1, 1), lambda i: (0, 0)),
            pl.BlockSpec((192, 128), lambda i: (0, 0)),
            pl.BlockSpec((192, 1), lambda i: (0, 0)),
            pl.BlockSpec((64, 1), lambda i: (0, 0)),
            pl.BlockSpec((64, 1), lambda i: (0, 0)),
            pl.BlockSpec((1, 1), lambda i: (0, 0)),
            pl.BlockSpec((192, 64), lambda i: (0, 0)),
            pl.BlockSpec((192, 1), lambda i: (0, 0)),
        ],
        out_specs=pl.BlockSpec((1, 192, n_p), lambda i: (i, 0, 0)),
        out_shape=jax.ShapeDtypeStruct((b, 192, n_p), jnp.float32),
        interpret=interpret,
    )(vox, zmax, zmin, zsum, zsq,
      vol_w1, vol_b1.reshape(128, 1), vol_g1.reshape(128, 1),
      vol_be1.reshape(128, 1), vol_a.reshape(1, 1),
      vol_w2, vol_b2.reshape(192, 1),
      knn_g.reshape(64, 1), knn_be.reshape(64, 1), knn_a.reshape(1, 1),
      out_w, out_b.reshape(192, 1))
    return out


def kernel(coords, coords2, fmap1, fmap2, vol_w1, vol_b1, vol_g1, vol_be1,
           vol_a, vol_w2, vol_b2, knn_w, knn_b, knn_g, knn_be, knn_a,
           out_w, out_b):
    return _pvx_forward(coords, coords2, fmap1, fmap2, vol_w1, vol_b1,
                        vol_g1, vol_be1, vol_a, vol_w2, vol_b2, knn_w, knn_b,
                        knn_g, knn_be, knn_a, out_w, out_b)


# speedup vs baseline: 1.0658x; 1.0658x over previous
import functools
import math

import jax
import jax.numpy as jnp
import numpy as np
from jax import lax
from jax.experimental import pallas as pl
from jax.experimental.pallas import tpu as pltpu
from jax.experimental.pallas import tpu_sc as plsc

_BIG = 1e30


def _bin_transform():
    p = np.array([[0.0, -0.5, 0.5], [1.0, 0.0, -1.0], [0.0, 0.5, 0.5]])
    t27 = np.kron(np.kron(p, p), p)
    t81 = np.zeros((81, 81), np.float32)
    for s in range(3):
        t81[27 * s:27 * (s + 1), 27 * s:27 * (s + 1)] = t27
    return t81


_T81 = _bin_transform()


def _pack_bf16_pair(a, b):
    ai = jax.lax.bitcast_convert_type(a, jnp.int32)
    bi = jax.lax.bitcast_convert_type(b, jnp.int32)
    ar = ai + 0x7FFF + ((ai >> 16) & 1)
    br = bi + 0x7FFF + ((bi >> 16) & 1)
    packed = (ar & np.int32(-65536)) | ((br >> 16) & 0xFFFF)
    return jax.lax.bitcast_convert_type(packed, jnp.float32)


def _unpack_bf16_pair(p):
    pi = jax.lax.bitcast_convert_type(p, jnp.int32)
    a = jax.lax.bitcast_convert_type(pi & np.int32(-65536), jnp.float32)
    b = jax.lax.bitcast_convert_type(pi << 16, jnp.float32)
    return a, b


def _corr_body(f1_ref, f2_ref, out_ref):
    d = f1_ref.shape[1]
    out_ref[0] = jax.lax.dot_general(
        f1_ref[0], f2_ref[0], (((0,), (0,)), ((), ())),
        preferred_element_type=jnp.float32,
        precision=jax.lax.Precision.HIGHEST) * (1.0 / math.sqrt(d))


def _round_rne(x):
    return (x + 12582912.0) - 12582912.0


def _sc_voxel_call(corr_nm, coords_flat, c2t_flat, rows, m):
    nw = 32
    rpw = rows // nw
    ch = 64
    nch = rpw // ch
    mesh = plsc.VectorSubcoreMesh(core_axis_name="c", subcore_axis_name="s")

    @functools.partial(
        pl.kernel, mesh=mesh,
        out_type=jax.ShapeDtypeStruct((rows, 96), jnp.float32),
        compiler_params=pltpu.CompilerParams(needs_layout_passes=False),
        scratch_types=[
            pltpu.VMEM((ch, m), jnp.float32),
            pltpu.VMEM((rpw * 3 + 16,), jnp.float32),
            pltpu.VMEM((3 * m,), jnp.float32),
            pltpu.VMEM((ch, 96), jnp.float32),
            pltpu.VMEM((96,), jnp.float32),
            pltpu.VMEM((96,), jnp.float32),
        ],
    )
    def k(corr_hbm, coords_hbm, c2_hbm, out_hbm, corr_v, cd_v, c2_v,
          ob_v, acc_v, cnt_v):
        wid = lax.axis_index("s") * 2 + lax.axis_index("c")
        bi = (wid * rpw) // (rows // 2)
        pltpu.sync_copy(coords_hbm.at[pl.ds(wid * rpw * 3, rpw * 3)],
                        cd_v.at[pl.ds(0, rpw * 3)])
        pltpu.sync_copy(c2_hbm.at[pl.ds(bi * 3 * m, 3 * m)], c2_v)
        ones16 = jnp.full((16,), 1.0, jnp.float32)
        zeros16 = jnp.zeros((16,), jnp.float32)
        for c6 in range(6):
            acc_v[pl.ds(16 * c6, 16)] = zeros16
            cnt_v[pl.ds(16 * c6, 16)] = zeros16

        def do_chunk(g, _):
            rowbase = wid * rpw + g * ch
            pltpu.sync_copy(corr_hbm.at[pl.ds(rowbase, ch), :], corr_v)

            def do_row(r, _):
                off = (g * ch + r) * 3
                cvec = cd_v[pl.ds(off, 16)]
                cx = jnp.full((16,), cvec[0], jnp.float32)
                cy = jnp.full((16,), cvec[1], jnp.float32)
                cz = jnp.full((16,), cvec[2], jnp.float32)

                def do_mchunk(j, _):
                    sl = pl.ds(j * 16, 16)
                    cj = corr_v[r, sl]
                    relx = c2_v[pl.ds(j * 16, 16)] - cx
                    rely = c2_v[pl.ds(m + j * 16, 16)] - cy
                    relz = c2_v[pl.ds(2 * m + j * 16, 16)] - cz
                    for si, s in enumerate((4.0, 2.0, 1.0)):
                        dvx = _round_rne(relx * s)
                        dvy = _round_rne(rely * s)
                        dvz = _round_rne(relz * s)
                        valid = ((jnp.abs(dvx) <= 1.0)
                                 & (jnp.abs(dvy) <= 1.0)
                                 & (jnp.abs(dvz) <= 1.0))
                        idxf = dvx * 9.0 + dvy * 3.0 + dvz + (13.0 + 27.0 * si)
                        idx = jnp.where(valid, idxf, 0.0).astype(jnp.int32)
                        plsc.addupdate_scatter(acc_v, [idx], cj, mask=valid)
                        plsc.addupdate_scatter(cnt_v, [idx], ones16,
                                               mask=valid)
                    return 0

                lax.fori_loop(0, m // 16, do_mchunk, 0)
                for c6 in range(6):
                    sl = pl.ds(16 * c6, 16)
                    a = acc_v[sl]
                    c = cnt_v[sl]
                    ob_v[r, sl] = a / jnp.maximum(c, 1.0)
                    acc_v[sl] = zeros16
                    cnt_v[sl] = zeros16
                return 0

            lax.fori_loop(0, ch, do_row, 0)
            pltpu.sync_copy(ob_v, out_hbm.at[pl.ds(rowbase, ch), :])
            return 0

        lax.fori_loop(0, nch, do_chunk, 0)

    return k(corr_nm, coords_flat, c2t_flat)


def _stage1_body(n_p2, knn, f1_ref, f2_ref, ct_ref, c2_ref, kw_ref, kb_ref,
                 zmax_ref, zmin_ref, zsum_ref, zsq_ref):
    f1 = f1_ref[0]
    f2 = f2_ref[0]
    d = f1.shape[0]
    n = f1.shape[1]
    m = n_p2
    corr = jax.lax.dot_general(
        f2, f1, (((0,), (0,)), ((), ())),
        preferred_element_type=jnp.float32,
        precision=jax.lax.Precision.HIGHEST) * (1.0 / math.sqrt(d))

    cx = ct_ref[0, 0:1, :]
    cy = ct_ref[0, 1:2, :]
    cz = ct_ref[0, 2:3, :]
    c2x = c2_ref[0, :, 0:1]
    c2y = c2_ref[0, :, 1:2]
    c2z = c2_ref[0, :, 2:3]
    relx = c2x - cx
    rely = c2y - cy
    relz = c2z - cz

    dist = relx * relx + rely * rely + relz * relz
    iota = jax.lax.broadcasted_iota(jnp.int32, (m, n), 0)
    key0 = ((jax.lax.bitcast_convert_type(dist, jnp.int32)
             & np.int32(-512)) | iota)
    p1 = _pack_bf16_pair(corr, relx)
    p2 = _pack_bf16_pair(rely, relz)
    kw0 = kw_ref[:, 0:1]
    kw1 = kw_ref[:, 1:2]
    kw2 = kw_ref[:, 2:3]
    kw3 = kw_ref[:, 3:4]
    kb = kb_ref[:, 0:1]

    def body(_, carry):
        kcur, zmax, zmin, fs1, fs2, fs3, fs4, zq = carry
        mn = jnp.min(kcur, axis=0, keepdims=True)
        fm = kcur == mn
        s1 = jnp.sum(jnp.where(fm, p1, 0.0), axis=0, keepdims=True)
        s2 = jnp.sum(jnp.where(fm, p2, 0.0), axis=0, keepdims=True)
        csel, rxs = _unpack_bf16_pair(s1)
        rys, rzs = _unpack_bf16_pair(s2)
        z = kw0 * csel + kw1 * rxs + kw2 * rys + kw3 * rzs + kb
        kcur = jnp.where(fm, jnp.int32(2 ** 31 - 1), kcur)
        return (kcur, jnp.maximum(zmax, z), jnp.minimum(zmin, z),
                fs1 + csel, fs2 + rxs, fs3 + rys, fs4 + rzs, zq + z * z)

    z0 = jnp.zeros((64, n), jnp.float32)
    r0 = jnp.zeros((1, n), jnp.float32)
    init = (key0, jnp.full((64, n), -_BIG, jnp.float32),
            jnp.full((64, n), _BIG, jnp.float32), r0, r0, r0, r0, z0)
    (_, zmax, zmin, fs1, fs2, fs3, fs4,
     zq) = jax.lax.fori_loop(0, knn, body, init)
    zs = (kw0 * fs1 + kw1 * fs2 + kw2 * fs3 + kw3 * fs4
          + jnp.float32(knn) * kb)
    zmax_ref[0] = zmax
    zmin_ref[0] = zmin

    ib = pl.program_id(1)
    ps = jnp.sum(zs, axis=1, keepdims=True)
    pq = jnp.sum(zq, axis=1, keepdims=True)

    @pl.when(ib == 0)
    def _():
        zsum_ref[0] = ps
        zsq_ref[0] = pq

    @pl.when(ib != 0)
    def _():
        zsum_ref[0] += ps
        zsq_ref[0] += pq


def _stage2_body(n_p, knn, vox_ref, zmax_ref, zmin_ref, zsum_ref, zsq_ref,
                 w1_ref, b1_ref, g1_ref, be1_ref, a1_ref, w2_ref, b2_ref,
                 kg_ref, kbe_ref, ka_ref, ow_ref, ob_ref, out_ref):
    eps = jnp.float32(1e-5)
    vox = vox_ref[0][:, 0:81]
    x = jax.lax.dot_general(
        w1_ref[...], vox, (((1,), (1,)), ((), ())),
        preferred_element_type=jnp.float32,
        precision=jax.lax.Precision.HIGHEST) + b1_ref[...]
    chs = jnp.sum(x, axis=1, keepdims=True)
    chq = jnp.sum(x * x, axis=1, keepdims=True)
    gi = jax.lax.broadcasted_iota(jnp.int32, (8, 128), 0)
    ci = jax.lax.broadcasted_iota(jnp.int32, (8, 128), 1)
    gmat = ((ci // 16) == gi).astype(jnp.float32)
    emat = gmat.T
    cnt1 = jnp.float32(16 * n_p)
    gmean = jax.lax.dot_general(gmat, chs, (((1,), (0,)), ((), ())),
                                preferred_element_type=jnp.float32) / cnt1
    gsq = jax.lax.dot_general(gmat, chq, (((1,), (0,)), ((), ())),
                              preferred_element_type=jnp.float32) / cnt1
    gvar = gsq - gmean * gmean
    mean_c = jax.lax.dot_general(emat, gmean, (((1,), (0,)), ((), ())),
                                 preferred_element_type=jnp.float32)
    var_c = jax.lax.dot_general(emat, gvar, (((1,), (0,)), ((), ())),
                                preferred_element_type=jnp.float32)
    xn = (x - mean_c) * (g1_ref[...] * jax.lax.rsqrt(var_c + eps)) + be1_ref[...]
    xa = jnp.where(xn >= 0.0, xn, a1_ref[0, 0] * xn)
    vol_out = jax.lax.dot_general(
        w2_ref[...], xa, (((1,), (0,)), ((), ())),
        preferred_element_type=jnp.float32,
        precision=jax.lax.Precision.HIGHEST) + b2_ref[...]

    zsum = zsum_ref[0]
    zsq = zsq_ref[0]
    gi2 = jax.lax.broadcasted_iota(jnp.int32, (8, 64), 0)
    ci2 = jax.lax.broadcasted_iota(jnp.int32, (8, 64), 1)
    gmat2 = ((ci2 // 8) == gi2).astype(jnp.float32)
    emat2 = gmat2.T
    cnt2 = jnp.float32(8 * n_p * knn)
    gmean2 = jax.lax.dot_general(gmat2, zsum, (((1,), (0,)), ((), ())),
                                 preferred_element_type=jnp.float32) / cnt2
    gsq2 = jax.lax.dot_general(gmat2, zsq, (((1,), (0,)), ((), ())),
                               preferred_element_type=jnp.float32) / cnt2
    gvar2 = gsq2 - gmean2 * gmean2
    mean_k = jax.lax.dot_general(emat2, gmean2, (((1,), (0,)), ((), ())),
                                 preferred_element_type=jnp.float32)
    var_k = jax.lax.dot_general(emat2, gvar2, (((1,), (0,)), ((), ())),
                                preferred_element_type=jnp.float32)
    slope = kg_ref[...] * jax.lax.rsqrt(var_k + eps)
    shift = kbe_ref[...] - mean_k * slope
    zpick = jnp.where(slope >= 0.0, zmax_ref[0], zmin_ref[0])
    kf = slope * zpick + shift
    kf = jnp.where(kf >= 0.0, kf, ka_ref[0, 0] * kf)
    knn_out = jax.lax.dot_general(
        ow_ref[...], kf, (((1,), (0,)), ((), ())),
        preferred_element_type=jnp.float32,
        precision=jax.lax.Precision.HIGHEST) + ob_ref[...]
    out_ref[0] = vol_out + knn_out


def _pvx_forward(coords, coords2, fmap1, fmap2, vol_w1, vol_b1, vol_g1,
                 vol_be1, vol_a, vol_w2, vol_b2, knn_w, knn_b, knn_g, knn_be,
                 knn_a, out_w, out_b, *, block_n=512, interpret=False):
    b, n_p, _ = coords.shape
    n_p2 = coords2.shape[1]
    d = fmap1.shape[1]
    knn = 32
    block_n = min(block_n, n_p)
    nb = n_p // block_n
    coords_t = jnp.transpose(coords, (0, 2, 1))
    kb2 = knn_b.reshape(64, 1)

    cb = 512
    corr_nm = pl.pallas_call(
        _corr_body,
        grid=(b, n_p // cb),
        in_specs=[
            pl.BlockSpec((1, d, cb), lambda i, j: (i, 0, j)),
            pl.BlockSpec((1, d, n_p2), lambda i, j: (i, 0, 0)),
        ],
        out_specs=pl.BlockSpec((1, cb, n_p2), lambda i, j: (i, j, 0)),
        out_shape=jax.ShapeDtypeStruct((b, n_p, n_p2), jnp.float32),
        interpret=interpret,
    )(fmap1, fmap2)

    rows = b * n_p
    vox = _sc_voxel_call(corr_nm.reshape(rows, n_p2),
                         coords.reshape(rows * 3),
                         jnp.transpose(coords2, (0, 2, 1)).reshape(b * 3 * n_p2),
                         rows, n_p2)
    vox = vox.reshape(b, n_p, 96)

    grid1 = (b, nb)
    zmax, zmin, zsum, zsq = pl.pallas_call(
        functools.partial(_stage1_body, n_p2, knn),
        grid=grid1,
        in_specs=[
            pl.BlockSpec((1, d, block_n), lambda i, j: (i, 0, j)),
            pl.BlockSpec((1, d, n_p2), lambda i, j: (i, 0, 0)),
            pl.BlockSpec((1, 3, block_n), lambda i, j: (i, 0, j)),
            pl.BlockSpec((1, n_p2, 3), lambda i, j: (i, 0, 0)),
            pl.BlockSpec((64, 4), lambda i, j: (0, 0)),
            pl.BlockSpec((64, 1), lambda i, j: (0, 0)),
        ],
        out_specs=[
            pl.BlockSpec((1, 64, block_n), lambda i, j: (i, 0, j)),
            pl.BlockSpec((1, 64, block_n), lambda i, j: (i, 0, j)),
            pl.BlockSpec((1, 64, 1), lambda i, j: (i, 0, 0)),
            pl.BlockSpec((1, 64, 1), lambda i, j: (i, 0, 0)),
        ],
        out_shape=[
            jax.ShapeDtypeStruct((b, 64, n_p), jnp.float32),
            jax.ShapeDtypeStruct((b, 64, n_p), jnp.float32),
            jax.ShapeDtypeStruct((b, 64, 1), jnp.float32),
            jax.ShapeDtypeStruct((b, 64, 1), jnp.float32),
        ],
        interpret=interpret,
    )(fmap1, fmap2, coords_t, coords2, knn_w, kb2)

    out = pl.pallas_call(
        functools.partial(_stage2_body, n_p, knn),
        grid=(b,),
        in_specs=[
            pl.BlockSpec((1, n_p, 96), lambda i: (i, 0, 0)),
            pl.BlockSpec((1, 64, n_p), lambda i: (i, 0, 0)),
            pl.BlockSpec((1, 64, n_p), lambda i: (i, 0, 0)),
            pl.BlockSpec((1, 64, 1), lambda i: (i, 0, 0)),
            pl.BlockSpec((1, 64, 1), lambda i: (i, 0, 0)),
            pl.BlockSpec((128, 81), lambda i: (0, 0)),
            pl.BlockSpec((128, 1), lambda i: (0, 0)),
            pl.BlockSpec((128, 1), lambda i: (0, 0)),
            pl.BlockSpec((128, 1), lambda i: (0, 0)),
            pl.BlockSpec((1, 1), lambda i: (0, 0)),
            pl.BlockSpec((192, 128), lambda i: (0, 0)),
            pl.BlockSpec((192, 1), lambda i: (0, 0)),
            pl.BlockSpec((64, 1), lambda i: (0, 0)),
            pl.BlockSpec((64, 1), lambda i: (0, 0)),
            pl.BlockSpec((1, 1), lambda i: (0, 0)),
            pl.BlockSpec((192, 64), lambda i: (0, 0)),
            pl.BlockSpec((192, 1), lambda i: (0, 0)),
        ],
        out_specs=pl.BlockSpec((1, 192, n_p), lambda i: (i, 0, 0)),
        out_shape=jax.ShapeDtypeStruct((b, 192, n_p), jnp.float32),
        interpret=interpret,
    )(vox, zmax, zmin, zsum, zsq,
      vol_w1, vol_b1.reshape(128, 1), vol_g1.reshape(128, 1),
      vol_be1.reshape(128, 1), vol_a.reshape(1, 1),
      vol_w2, vol_b2.reshape(192, 1),
      knn_g.reshape(64, 1), knn_be.reshape(64, 1), knn_a.reshape(1, 1),
      out_w, out_b.reshape(192, 1))
    return out


def kernel(coords, coords2, fmap1, fmap2, vol_w1, vol_b1, vol_g1, vol_be1,
           vol_a, vol_w2, vol_b2, knn_w, knn_b, knn_g, knn_be, knn_a,
           out_w, out_b):
    return _pvx_forward(coords, coords2, fmap1, fmap2, vol_w1, vol_b1,
                        vol_g1, vol_be1, vol_a, vol_w2, vol_b2, knn_w, knn_b,
                        knn_g, knn_be, knn_a, out_w, out_b)
